# Initial kernel scaffold; baseline (speedup 1.0000x reference)
import jax
import jax.numpy as jnp
from jax.experimental import pallas as pl


def kernel(grids, rgb2gray_weight, grid_xy, rgb, idx):
    raise NotImplementedError("write your pallas kernel here")


# speedup vs baseline: 1.7163x; 1.7163x over previous
import functools

import jax
import jax.numpy as jnp
from jax import lax
from jax.experimental import pallas as pl
from jax.experimental.pallas import tpu as pltpu
from jax.experimental.pallas import tpu_sc as plsc

NUM = 1000
W = 16
H = 16
L = 8
C = 12
CP = 16
T = 1048576

NC = 2
NS = 16
NW = NC * NS
PW = T // NW
K = 128
NCHUNK = PW // K
ROW_PAD = 288
TROWS = NUM * L * H * W + ROW_PAD

DK = (0, 1, W, W + 1, H * W, H * W + 1, H * W + W, H * W + W + 1)


def _body(table, xyf, rgbf, idx_h, w8, out_h,
          xb, yb, rb, gb, bb, ib, wv, idxb, wtb, rows, ob, sem):
    wid = lax.axis_index("s") * NC + lax.axis_index("c")
    base_w = wid * PW

    pltpu.sync_copy(w8, wv)
    wvv = wv[...]
    w0 = wvv[0]
    w1 = wvv[1]
    w2 = wvv[2]

    def chunk_body(c, _):
        p = base_w + c * K
        pltpu.sync_copy(xyf.at[pl.ds(p, K)], xb)
        pltpu.sync_copy(xyf.at[pl.ds(T + p, K)], yb)
        pltpu.sync_copy(rgbf.at[pl.ds(p, K)], rb)
        pltpu.sync_copy(rgbf.at[pl.ds(T + p, K)], gb)
        pltpu.sync_copy(rgbf.at[pl.ds(2 * T + p, K)], bb)
        pltpu.sync_copy(idx_h.at[pl.ds(p, K)], ib)

        def coord_body(g, _):
            s = g * 16
            xv = xb[pl.ds(s, 16)]
            yv = yb[pl.ds(s, 16)]
            rv = rb[pl.ds(s, 16)]
            gv = gb[pl.ds(s, 16)]
            bv = bb[pl.ds(s, 16)]
            iv = ib[pl.ds(s, 16)]
            zraw = (rv * w0 + gv * w1 + bv * w2) * 2.0 - 1.0
            x2 = xv * 2.0 - 1.0
            y2 = yv * 2.0 - 1.0
            ix = jnp.clip((x2 + 1.0) * (0.5 * (W - 1)), 0.0, W - 1)
            iy = jnp.clip((y2 + 1.0) * (0.5 * (H - 1)), 0.0, H - 1)
            iz = jnp.clip((zraw + 1.0) * (0.5 * (L - 1)), 0.0, L - 1)
            x0 = ix.astype(jnp.int32)
            y0 = iy.astype(jnp.int32)
            z0 = iz.astype(jnp.int32)
            wx = ix - x0.astype(jnp.float32)
            wy = iy - y0.astype(jnp.float32)
            wz = iz - z0.astype(jnp.float32)
            ux = 1.0 - wx
            uy = 1.0 - wy
            uz = 1.0 - wz
            lin = iv * (L * H * W) + z0 * (H * W) + y0 * W + x0
            for k in range(8):
                idxb[k, pl.ds(s, 16)] = lin + DK[k]
            zy00 = uz * uy
            zy01 = uz * wy
            zy10 = wz * uy
            zy11 = wz * wy
            wtb[0, pl.ds(s, 16)] = zy00 * ux
            wtb[1, pl.ds(s, 16)] = zy00 * wx
            wtb[2, pl.ds(s, 16)] = zy01 * ux
            wtb[3, pl.ds(s, 16)] = zy01 * wx
            wtb[4, pl.ds(s, 16)] = zy10 * ux
            wtb[5, pl.ds(s, 16)] = zy10 * wx
            wtb[6, pl.ds(s, 16)] = zy11 * ux
            wtb[7, pl.ds(s, 16)] = zy11 * wx
            return 0

        lax.fori_loop(0, K // 16, coord_body, 0, unroll=False)

        for k in range(8):
            pltpu.async_copy(table.at[idxb.at[k]], rows.at[k], sem).wait()

        def blend_body(g, _):
            s = g * 16
            t = lax.iota(jnp.int32, 16) + s
            rv = rb[pl.ds(s, 16)]
            gv = gb[pl.ds(s, 16)]
            bv = bb[pl.ds(s, 16)]
            wk = [wtb[k, pl.ds(s, 16)] for k in range(8)]
            v = []
            for ch in range(C):
                cs = jnp.full((16,), ch, jnp.int32)
                acc = None
                for k in range(8):
                    ks = jnp.full((16,), k, jnp.int32)
                    g8 = plsc.load_gather(rows, [ks, t, cs])
                    acc = g8 * wk[k] if acc is None else acc + g8 * wk[k]
                v.append(acc)
            o0 = v[0] * rv + v[1] * gv + v[2] * bv + v[3]
            o1 = v[4] * rv + v[5] * gv + v[6] * bv + v[7]
            o2 = v[8] * rv + v[9] * gv + v[10] * bv + v[11]
            ob[0, pl.ds(s, 16)] = o0
            ob[1, pl.ds(s, 16)] = o1
            ob[2, pl.ds(s, 16)] = o2
            return 0

        lax.fori_loop(0, K // 16, blend_body, 0, unroll=False)

        pltpu.sync_copy(ob.at[0], out_h.at[pl.ds(p, K)])
        pltpu.sync_copy(ob.at[1], out_h.at[pl.ds(T + p, K)])
        pltpu.sync_copy(ob.at[2], out_h.at[pl.ds(2 * T + p, K)])
        return 0

    lax.fori_loop(0, NCHUNK, chunk_body, 0, unroll=False)


@jax.jit
def kernel(grids, rgb2gray_weight, grid_xy, rgb, idx):
    gf = jnp.transpose(grids, (0, 2, 3, 4, 1)).reshape(-1, C)
    table = jnp.pad(gf, ((0, ROW_PAD), (0, CP - C)))
    xyf = grid_xy.T.reshape(-1)
    rgbf = rgb.T.reshape(-1)
    w8 = jnp.pad(rgb2gray_weight.reshape(-1), (0, 13))

    mesh = plsc.VectorSubcoreMesh(
        core_axis_name="c", subcore_axis_name="s",
        num_cores=NC, num_subcores=NS)
    run = pl.kernel(
        _body,
        out_type=jax.ShapeDtypeStruct((3 * T,), jnp.float32),
        mesh=mesh,
        scratch_types=[
            pltpu.VMEM((K,), jnp.float32),
            pltpu.VMEM((K,), jnp.float32),
            pltpu.VMEM((K,), jnp.float32),
            pltpu.VMEM((K,), jnp.float32),
            pltpu.VMEM((K,), jnp.float32),
            pltpu.VMEM((K,), jnp.int32),
            pltpu.VMEM((16,), jnp.float32),
            pltpu.VMEM((8, K), jnp.int32),
            pltpu.VMEM((8, K), jnp.float32),
            pltpu.VMEM((8, K, CP), jnp.float32),
            pltpu.VMEM((3, K), jnp.float32),
            pltpu.SemaphoreType.DMA,
        ],
        compiler_params=pltpu.CompilerParams(
            needs_layout_passes=False, use_tc_tiling_on_sc=False),
    )
    out3 = run(table, xyf, rgbf, idx, w8)
    return out3.reshape(3, T).T
